# baseline (speedup 1.0000x reference)
import functools

import jax
import jax.numpy as jnp
from jax import lax
from jax.experimental import pallas as pl
from jax.experimental.pallas import tpu as pltpu
from jax.experimental.pallas import tpu_sc as plsc

D = 64
L = 16
NCH = D // L
N_TOTAL = 32768
NC, NS = 2, 16
NW = NC * NS
PER_W = N_TOTAL // NW
CHUNK = 128
NCHUNKS = PER_W // CHUNK
GPC = CHUNK // L
IDX_ROW = 128
VHALF = 500000

_GDN = lax.GatherDimensionNumbers(
    offset_dims=(), collapsed_slice_dims=(0,), start_index_map=(0,))


def _lane_perm(v, perm):
    return lax.gather(v, perm, _GDN, slice_sizes=(1,),
                      mode=lax.GatherScatterMode.PROMISE_IN_BOUNDS)


def _body(z_hbm, idxi_hbm, idxj_hbm, w2t_hbm, w3_hbm, out_hbm,
          idxi_v, idxj_v, gidxi_v, gidxj_v,
          zi0, zj0, zi1, zj1, out_v, w2t_v, w3_v, sem):
    wid = lax.axis_index("s") * NC + lax.axis_index("c")

    iota = lax.iota(jnp.int32, L)
    perms = [(iota ^ (1 << k)).reshape(L, 1) for k in range(4)]
    masks = [iota == r for r in range(L)]

    irow0 = wid * (PER_W // IDX_ROW)
    pltpu.sync_copy(idxi_hbm.at[pl.ds(irow0, PER_W // IDX_ROW)], idxi_v)
    pltpu.sync_copy(idxj_hbm.at[pl.ds(irow0, PER_W // IDX_ROW)], idxj_v)
    for k in range(PER_W // IDX_ROW):
        for v in range(IDX_ROW // L):
            sl = pl.ds(v * L, L)
            gidxi_v[k, sl] = idxi_v[k, sl] >> 1
            gidxj_v[k, sl] = idxj_v[k, sl] >> 1

    zbufs = [(zi0, zj0), (zi1, zj1)]

    def fire(c):
        zi, zj = zbufs[c % 2]
        return [
            pltpu.async_copy(z_hbm.at[gidxi_v.at[c]], zi, sem),
            pltpu.async_copy(z_hbm.at[gidxj_v.at[c]], zj, sem),
        ]

    pending = fire(0)

    pltpu.sync_copy(w2t_hbm, w2t_v)
    pltpu.sync_copy(w3_hbm, w3_v)
    u_chunks = None
    for ccv in range(NCH):
        w3a_vec = w3_v[pl.ds(ccv * L, L)]
        for e in range(L):
            s = w3a_vec[e]
            cc = ccv * L + e
            if u_chunks is None:
                u_chunks = [w2t_v[cc, pl.ds(kc * L, L)] * s
                            for kc in range(2 * NCH)]
            else:
                for kc in range(2 * NCH):
                    u_chunks[kc] = (u_chunks[kc]
                                    + w2t_v[cc, pl.ds(kc * L, L)] * s)
    ua = u_chunks[:NCH]
    ub = u_chunks[NCH:]
    w3b = [w3_v[pl.ds(D + cc * L, L)] for cc in range(NCH)]

    one = jnp.float32(1.0)
    zero = jnp.float32(0.0)

    for c in range(NCHUNKS):
        for h in pending:
            h.wait()
        if c + 1 < NCHUNKS:
            pending = fire(c + 1)
        zi, zj = zbufs[c % 2]

        def group(g, carry, zi=zi, zj=zj, c=c):
            gsl = pl.ds(g * L, L)
            basei = (idxi_v[c, gsl] & 1) * D
            basej = (idxj_v[c, gsl] & 1) * D
            sums = []
            for r in range(L):
                row = g * L + r
                bi = basei[r]
                bj = basej[r]
                acc = None
                for cc in range(NCH):
                    a = zi[row, pl.ds(bi + cc * L, L)]
                    b = zj[row, pl.ds(bj + cc * L, L)]
                    t = (jnp.maximum(a, zero) * ua[cc]
                         + jnp.maximum(b, zero) * ub[cc]
                         + (a * b) * w3b[cc])
                    acc = t if acc is None else acc + t
                for p in perms:
                    acc = acc + _lane_perm(acc, p)
                sums.append(jnp.where(masks[r], acc, zero))
            while len(sums) > 1:
                sums = [sums[i] + sums[i + 1] for i in range(0, len(sums), 2)]
            sig = one / (one + jnp.exp(-sums[0]))
            out_v[pl.ds(c * CHUNK + g * L, L)] = sig
            return carry

        lax.fori_loop(0, GPC, group, jnp.int32(0))

    pltpu.sync_copy(out_v, out_hbm.at[pl.ds(wid * PER_W, PER_W)])


@jax.jit
def _run(z2, idxi, idxj, w2t, w3flat):
    mesh = plsc.VectorSubcoreMesh(core_axis_name="c", subcore_axis_name="s")
    k = functools.partial(
        pl.kernel,
        mesh=mesh,
        compiler_params=pltpu.CompilerParams(use_tc_tiling_on_sc=True),
        out_type=jax.ShapeDtypeStruct((N_TOTAL,), jnp.float32),
        scratch_types=[
            pltpu.VMEM((PER_W // IDX_ROW, IDX_ROW), jnp.int32),
            pltpu.VMEM((PER_W // IDX_ROW, IDX_ROW), jnp.int32),
            pltpu.VMEM((PER_W // IDX_ROW, IDX_ROW), jnp.int32),
            pltpu.VMEM((PER_W // IDX_ROW, IDX_ROW), jnp.int32),
            pltpu.VMEM((CHUNK, 2 * D), jnp.float32),
            pltpu.VMEM((CHUNK, 2 * D), jnp.float32),
            pltpu.VMEM((CHUNK, 2 * D), jnp.float32),
            pltpu.VMEM((CHUNK, 2 * D), jnp.float32),
            pltpu.VMEM((PER_W,), jnp.float32),
            pltpu.VMEM((D, 2 * D), jnp.float32),
            pltpu.VMEM((2 * D,), jnp.float32),
            pltpu.SemaphoreType.DMA,
        ],
    )(_body)
    return k(z2, idxi, idxj, w2t, w3flat)


def kernel(X, train_edges, train_false_edges, z, weight_two, weight_three):
    edges = jnp.concatenate([train_edges, train_false_edges], axis=0)
    idxi = edges[:, 0].reshape(N_TOTAL // IDX_ROW, IDX_ROW)
    idxj = edges[:, 1].reshape(N_TOTAL // IDX_ROW, IDX_ROW)
    z2 = z.reshape(VHALF, 2 * D)
    out = _run(z2, idxi, idxj, weight_two.T, weight_three.reshape(2 * D))
    return out.reshape(N_TOTAL, 1)


# speedup vs baseline: 1.6501x; 1.6501x over previous
import functools

import jax
import jax.numpy as jnp
from jax import lax
from jax.experimental import pallas as pl
from jax.experimental.pallas import tpu as pltpu
from jax.experimental.pallas import tpu_sc as plsc

D = 64
L = 16
NCH = D // L
N_TOTAL = 32768
NC, NS = 2, 16
NW = NC * NS
PER_W = N_TOTAL // NW
CHUNK = 128
NCHUNKS = PER_W // CHUNK
GPC = CHUNK // L
IDX_ROW = 128

_GDN = lax.GatherDimensionNumbers(
    offset_dims=(), collapsed_slice_dims=(0,), start_index_map=(0,))


def _lane_perm(v, perm):
    return lax.gather(v, perm, _GDN, slice_sizes=(1,),
                      mode=lax.GatherScatterMode.PROMISE_IN_BOUNDS)


def _body(z_hbm, idxi_hbm, idxj_hbm, w2t_hbm, w3_hbm, out_hbm,
          idxi_v, idxj_v, zi0, zj0, zi1, zj1, out_v, w2t_v, w3_v,
          sem0, sem1):
    wid = lax.axis_index("s") * NC + lax.axis_index("c")

    iota = lax.iota(jnp.int32, L)
    perms = [(iota ^ (1 << k)).reshape(L, 1) for k in range(4)]
    masks = [iota == r for r in range(L)]

    irow0 = wid * (PER_W // IDX_ROW)
    pltpu.sync_copy(idxi_hbm.at[pl.ds(irow0, PER_W // IDX_ROW)], idxi_v)
    pltpu.sync_copy(idxj_hbm.at[pl.ds(irow0, PER_W // IDX_ROW)], idxj_v)

    def fire(c, zi, zj, sem):
        def grp(g, carry):
            ivec = idxi_v[c, pl.ds(g * L, L)]
            jvec = idxj_v[c, pl.ds(g * L, L)]
            for r in range(L):
                pltpu.async_copy(
                    z_hbm.at[pl.ds(ivec[r], 1)],
                    zi.at[pl.ds(g * L + r, 1)], sem)
                pltpu.async_copy(
                    z_hbm.at[pl.ds(jvec[r], 1)],
                    zj.at[pl.ds(g * L + r, 1)], sem)
            return carry

        lax.fori_loop(0, GPC, grp, jnp.int32(0))

    def drain(zi, zj, sem):
        pltpu.make_async_copy(z_hbm.at[pl.ds(0, CHUNK)], zi, sem).wait()
        pltpu.make_async_copy(z_hbm.at[pl.ds(0, CHUNK)], zj, sem).wait()

    fire(0, zi0, zj0, sem0)

    pltpu.sync_copy(w2t_hbm, w2t_v)
    pltpu.sync_copy(w3_hbm, w3_v)
    u_chunks = None
    for ccv in range(NCH):
        w3a_vec = w3_v[pl.ds(ccv * L, L)]
        for e in range(L):
            s = w3a_vec[e]
            cc = ccv * L + e
            if u_chunks is None:
                u_chunks = [w2t_v[cc, pl.ds(kc * L, L)] * s
                            for kc in range(2 * NCH)]
            else:
                for kc in range(2 * NCH):
                    u_chunks[kc] = (u_chunks[kc]
                                    + w2t_v[cc, pl.ds(kc * L, L)] * s)
    ua = u_chunks[:NCH]
    ub = u_chunks[NCH:]
    w3b = [w3_v[pl.ds(D + cc * L, L)] for cc in range(NCH)]

    one = jnp.float32(1.0)
    zero = jnp.float32(0.0)

    def compute(c, zi, zj):
        def group(g, carry):
            sums = []
            for r in range(L):
                row = g * L + r
                acc = None
                for cc in range(NCH):
                    a = zi[row, pl.ds(cc * L, L)]
                    b = zj[row, pl.ds(cc * L, L)]
                    t = (jnp.maximum(a, zero) * ua[cc]
                         + jnp.maximum(b, zero) * ub[cc]
                         + (a * b) * w3b[cc])
                    acc = t if acc is None else acc + t
                for p in perms:
                    acc = acc + _lane_perm(acc, p)
                sums.append(jnp.where(masks[r], acc, zero))
            while len(sums) > 1:
                sums = [sums[i] + sums[i + 1] for i in range(0, len(sums), 2)]
            sig = one / (one + jnp.exp(-sums[0]))
            out_v[pl.ds(c * CHUNK + g * L, L)] = sig
            return carry

        lax.fori_loop(0, GPC, group, jnp.int32(0))

    def pair(t, carry):
        c0 = 2 * t
        c1 = c0 + 1
        fire(c1, zi1, zj1, sem1)
        drain(zi0, zj0, sem0)
        compute(c0, zi0, zj0)

        @pl.when(t + 1 < NCHUNKS // 2)
        def _():
            fire(c0 + 2, zi0, zj0, sem0)

        drain(zi1, zj1, sem1)
        compute(c1, zi1, zj1)
        return carry

    lax.fori_loop(0, NCHUNKS // 2, pair, jnp.int32(0))

    pltpu.sync_copy(out_v, out_hbm.at[pl.ds(wid * PER_W, PER_W)])


@jax.jit
def _run(z, idxi, idxj, w2t, w3flat):
    mesh = plsc.VectorSubcoreMesh(core_axis_name="c", subcore_axis_name="s")
    k = functools.partial(
        pl.kernel,
        mesh=mesh,
        compiler_params=pltpu.CompilerParams(use_tc_tiling_on_sc=True),
        out_type=jax.ShapeDtypeStruct((N_TOTAL,), jnp.float32),
        scratch_types=[
            pltpu.VMEM((PER_W // IDX_ROW, IDX_ROW), jnp.int32),
            pltpu.VMEM((PER_W // IDX_ROW, IDX_ROW), jnp.int32),
            pltpu.VMEM((CHUNK, D), jnp.float32),
            pltpu.VMEM((CHUNK, D), jnp.float32),
            pltpu.VMEM((CHUNK, D), jnp.float32),
            pltpu.VMEM((CHUNK, D), jnp.float32),
            pltpu.VMEM((PER_W,), jnp.float32),
            pltpu.VMEM((D, 2 * D), jnp.float32),
            pltpu.VMEM((2 * D,), jnp.float32),
            pltpu.SemaphoreType.DMA,
            pltpu.SemaphoreType.DMA,
        ],
    )(_body)
    return k(z, idxi, idxj, w2t, w3flat)


def kernel(X, train_edges, train_false_edges, z, weight_two, weight_three):
    edges = jnp.concatenate([train_edges, train_false_edges], axis=0)
    idxi = edges[:, 0].reshape(N_TOTAL // IDX_ROW, IDX_ROW)
    idxj = edges[:, 1].reshape(N_TOTAL // IDX_ROW, IDX_ROW)
    out = _run(z, idxi, idxj, weight_two.T, weight_three.reshape(2 * D))
    return out.reshape(N_TOTAL, 1)
